# baseline (speedup 1.0000x reference)
import functools

import jax
import jax.numpy as jnp
from jax import lax
from jax.experimental import pallas as pl
from jax.experimental.pallas import tpu as pltpu
from jax.experimental.pallas import tpu_sc as plsc

OUT_H, OUT_W = 7, 7
NB = OUT_H * OUT_W
SPATIAL_SCALE = 0.25
B, C, H, W = 2, 256, 200, 200
N_ROIS = 1000
N_PAD = 1024
K_ROWS = 4 * NB
K_PAD = 224
K_G2 = 96
K_GATH = 112 + K_G2
CV = C // 16
OUT_ROW = C * NB


def _prep_body(rois_ref, masks_ref, i00, i01, i10, i11, w00, w01, w10, w11):
    batch = rois_ref[0].astype(jnp.int32)
    x1 = rois_ref[1] * SPATIAL_SCALE
    y1 = rois_ref[2] * SPATIAL_SCALE
    x2 = rois_ref[3] * SPATIAL_SCALE
    y2 = rois_ref[4] * SPATIAL_SCALE
    bin_w = jnp.maximum(x2 - x1, 1.0) * (1.0 / OUT_W)
    bin_h = jnp.maximum(y2 - y1, 1.0) * (1.0 / OUT_H)

    b = lax.broadcasted_iota(jnp.int32, (N_PAD, NB), 1)
    jj = (b % OUT_W).astype(jnp.float32) + 0.5
    ji = (b // OUT_W).astype(jnp.float32) + 0.5
    x = x1[:, None] + jj * bin_w[:, None]
    y = y1[:, None] + ji * bin_h[:, None]
    x0f = jnp.floor(x)
    y0f = jnp.floor(y)
    lx = x - x0f
    ly = y - y0f
    hx = 1.0 - lx
    hy = 1.0 - ly
    x0 = jnp.clip(x0f, 0, W - 1).astype(jnp.int32)
    x1i = jnp.clip(x0f + 1.0, 0, W - 1).astype(jnp.int32)
    y0 = jnp.clip(y0f, 0, H - 1).astype(jnp.int32)
    y1i = jnp.clip(y0f + 1.0, 0, H - 1).astype(jnp.int32)
    base = (batch * (H * W))[:, None]
    i00[...] = base + y0 * W + x0
    i01[...] = base + y0 * W + x1i
    i10[...] = base + y1i * W + x0
    i11[...] = base + y1i * W + x1i
    m = masks_ref[...]
    w00[...] = hy * hx * m
    w01[...] = hy * lx * m
    w10[...] = ly * hx * m
    w11[...] = ly * lx * m


def _prep(rois_t, masks_p):
    shp_i = jax.ShapeDtypeStruct((N_PAD, NB), jnp.int32)
    shp_f = jax.ShapeDtypeStruct((N_PAD, NB), jnp.float32)
    return pl.pallas_call(
        _prep_body,
        out_shape=[shp_i] * 4 + [shp_f] * 4,
    )(rois_t, masks_p)


def _sc_body(table_hbm, idx_hbm, w_hbm, out_hbm, idx_v, w_v, rows_v, out_v, sem):
    nc = 2
    wid = lax.axis_index("s") * nc + lax.axis_index("c")
    r0 = wid * (N_PAD // 32)

    lane = lax.iota(jnp.int32, 16)
    cvecs = [(lane + cb * 16) * NB for cb in range(CV)]

    def roi_body(i, carry):
        r = r0 + i

        @pl.when(r < N_ROIS)
        def _():
            pltpu.sync_copy(idx_hbm.at[r], idx_v)
            pltpu.sync_copy(w_hbm.at[r], w_v)
            cp1 = pltpu.async_copy(
                table_hbm.at[idx_v.at[pl.ds(0, 112)]],
                rows_v.at[pl.ds(0, 112)], sem)
            cp2 = pltpu.async_copy(
                table_hbm.at[idx_v.at[pl.ds(112, K_G2)]],
                rows_v.at[pl.ds(112, K_G2)], sem)
            cp1.wait()
            cp2.wait()

            def bin_body(j, c2):
                jf = jnp.full((16,), j, jnp.int32)
                wv0 = plsc.load_gather(w_v, [jf])
                wv1 = plsc.load_gather(w_v, [jf + NB])
                wv2 = plsc.load_gather(w_v, [jf + 2 * NB])
                wv3 = plsc.load_gather(w_v, [jf + 3 * NB])
                for cb in range(CV):
                    sl = pl.ds(cb * 16, 16)
                    acc = wv0 * rows_v[j, sl]
                    acc = acc + wv1 * rows_v[j + NB, sl]
                    acc = acc + wv2 * rows_v[j + 2 * NB, sl]
                    acc = acc + wv3 * rows_v[j + 3 * NB, sl]
                    plsc.store_scatter(out_v, [cvecs[cb] + jf], acc)
                return c2

            lax.fori_loop(0, NB, bin_body, 0)
            pltpu.sync_copy(out_v, out_hbm.at[r])

        return carry

    lax.fori_loop(0, N_PAD // 32, roi_body, 0)


def kernel(inputs, rois, masks):
    flat = jnp.transpose(inputs, (0, 2, 3, 1)).reshape(B * H * W, C)

    rois_t = jnp.pad(rois, ((0, N_PAD - N_ROIS), (0, 0))).T
    masks_p = jnp.pad(masks.reshape(N_ROIS, NB),
                      ((0, N_PAD - N_ROIS), (0, 0)))

    i00, i01, i10, i11, w00, w01, w10, w11 = _prep(rois_t, masks_p)
    idx_all = jnp.stack([i00, i01, i10, i11], axis=1).reshape(N_PAD, K_ROWS)
    idx_all = jnp.pad(idx_all, ((0, 0), (0, K_PAD - K_ROWS)))
    w_all = jnp.stack([w00, w01, w10, w11], axis=1).reshape(N_PAD, K_ROWS)
    w_all = jnp.pad(w_all, ((0, 0), (0, K_PAD - K_ROWS)))

    mesh = plsc.VectorSubcoreMesh(core_axis_name="c", subcore_axis_name="s")
    sc_fn = pl.kernel(
        _sc_body,
        out_type=jax.ShapeDtypeStruct((N_ROIS, OUT_ROW), jnp.float32),
        mesh=mesh,
        scratch_types=[
            pltpu.VMEM((K_PAD,), jnp.int32),
            pltpu.VMEM((K_PAD,), jnp.float32),
            pltpu.VMEM((K_GATH, C), jnp.float32),
            pltpu.VMEM((OUT_ROW,), jnp.float32),
            pltpu.SemaphoreType.DMA,
        ],
        compiler_params=pltpu.CompilerParams(needs_layout_passes=False),
    )
    out = sc_fn(flat, idx_all, w_all)
    return out.reshape(N_ROIS, C, OUT_H, OUT_W)
